# baseline (speedup 1.0000x reference)
import jax
import jax.numpy as jnp
from jax.experimental import pallas as pl
from jax.experimental.pallas import tpu as pltpu

_NCHUNK = 8


def _dma_body(kv_ref, vv_ref, kc_ref, vc_ref, ko_ref, vo_ref, sems):
    n = kc_ref.shape[0]
    ch = n // _NCHUNK
    bulk = []
    i = 0
    for src, dst in ((kc_ref, ko_ref), (vc_ref, vo_ref)):
        for c in range(_NCHUNK):
            sl = pl.ds(c * ch, ch)
            bulk.append(pltpu.make_async_copy(src.at[sl], dst.at[sl], sems.at[i]))
            i += 1
    for c in bulk:
        c.start()
    for c in bulk:
        c.wait()
    fills = []
    for val, dst in ((kv_ref, ko_ref), (vv_ref, vo_ref)):
        fills.append(pltpu.make_async_copy(
            val, dst.at[:, pl.ds(0, val.shape[1]), :], sems.at[i]))
        i += 1
    for c in fills:
        c.start()
    for c in fills:
        c.wait()


def kernel(input_pos, k_val, v_val, k_cache, v_cache, pos):
    B, H, S_new, D = k_val.shape
    L = k_cache.shape[2]
    BH = B * H
    kc = k_cache.reshape(BH, L, D)
    vc = v_cache.reshape(BH, L, D)
    kv = k_val.reshape(BH, S_new, D)
    vv = v_val.reshape(BH, S_new, D)

    hbm = pl.BlockSpec(memory_space=pltpu.MemorySpace.HBM)
    nsem = 2 * _NCHUNK + 2
    ko, vo = pl.pallas_call(
        _dma_body,
        in_specs=[hbm, hbm, hbm, hbm],
        out_specs=[hbm, hbm],
        out_shape=[
            jax.ShapeDtypeStruct((BH, L, D), k_cache.dtype),
            jax.ShapeDtypeStruct((BH, L, D), v_cache.dtype),
        ],
        scratch_shapes=[pltpu.SemaphoreType.DMA((nsem,))],
    )(kv, vv, kc, vc)
    return (ko.reshape(B, H, L, D), vo.reshape(B, H, L, D))


# speedup vs baseline: 47.3984x; 47.3984x over previous
import jax
import jax.numpy as jnp
from jax.experimental import pallas as pl
from jax.experimental.pallas import tpu as pltpu

_CH = 16


def _fill_body(kv_ref, vv_ref, kc_ref, vc_ref, ko_ref, vo_ref):
    ko_ref[...] = kv_ref[...]
    vo_ref[...] = vv_ref[...]


def kernel(input_pos, k_val, v_val, k_cache, v_cache, pos):
    B, H, S_new, D = k_val.shape
    L = k_cache.shape[2]
    BH = B * H
    kc = k_cache.reshape(BH, L, D)
    vc = v_cache.reshape(BH, L, D)
    kv = k_val.reshape(BH, S_new, D)
    vv = v_val.reshape(BH, S_new, D)

    val_spec = pl.BlockSpec((_CH, S_new, D), lambda i: (i, 0, 0))
    hbm = pl.BlockSpec(memory_space=pltpu.MemorySpace.HBM)
    out_spec = pl.BlockSpec((_CH, S_new, D), lambda i: (i, 0, 0))

    ko, vo = pl.pallas_call(
        _fill_body,
        grid=(BH // _CH,),
        in_specs=[val_spec, val_spec, hbm, hbm],
        out_specs=[out_spec, out_spec],
        out_shape=[
            jax.ShapeDtypeStruct((BH, L, D), k_cache.dtype),
            jax.ShapeDtypeStruct((BH, L, D), v_cache.dtype),
        ],
        input_output_aliases={2: 0, 3: 1},
    )(kv, vv, kc, vc)
    return (ko.reshape(B, H, L, D), vo.reshape(B, H, L, D))
